# baseline (speedup 1.0000x reference)
import jax
import jax.numpy as jnp
from jax.experimental import pallas as pl
from jax.experimental.pallas import tpu as pltpu


def _dma_kernel(emb_ref, out_ref, sem):
    copy = pltpu.make_async_copy(emb_ref, out_ref, sem)
    copy.start()
    copy.wait()


def kernel(inputs, embedding):
    out = pl.pallas_call(
        _dma_kernel,
        in_specs=[pl.BlockSpec(memory_space=pl.ANY)],
        out_specs=pl.BlockSpec(memory_space=pl.ANY),
        out_shape=jax.ShapeDtypeStruct(embedding.shape, embedding.dtype),
        scratch_shapes=[pltpu.SemaphoreType.DMA],
    )(embedding)
    return out[None]


# speedup vs baseline: 20.4211x; 20.4211x over previous
import functools

import jax
import jax.numpy as jnp
from jax import lax
from jax.experimental import pallas as pl
from jax.experimental.pallas import tpu as pltpu
from jax.experimental.pallas import tpu_sc as plsc


def _sc_copy_body(nc, rows_per_w, chunk, nchunks, emb_hbm, out_hbm,
                  b0, b1, ls0, ls1, ss0, ss1):
    wid = lax.axis_index("s") * nc + lax.axis_index("c")
    base = wid * rows_per_w
    bufs = (b0, b1)
    lsems = (ls0, ls1)
    ssems = (ss0, ss1)

    def src(i):
        return emb_hbm.at[pl.ds(base + i * chunk, chunk)]

    def dst(i):
        return out_hbm.at[pl.ds(base + i * chunk, chunk)]

    loads = {}
    stores = {}
    loads[0] = pltpu.async_copy(src(0), bufs[0], lsems[0])
    if nchunks > 1:
        loads[1] = pltpu.async_copy(src(1), bufs[1], lsems[1])
    for i in range(nchunks):
        b = i % 2
        loads[i].wait()
        stores[i] = pltpu.async_copy(bufs[b], dst(i), ssems[b])
        if i + 2 < nchunks:
            stores[i].wait()
            loads[i + 2] = pltpu.async_copy(src(i + 2), bufs[b], lsems[b])
    for i in range(max(0, nchunks - 2), nchunks):
        stores[i].wait()


def kernel(inputs, embedding):
    seq_len, embd_dim = embedding.shape
    mesh = plsc.VectorSubcoreMesh(core_axis_name="c", subcore_axis_name="s")
    nw = mesh.num_cores * mesh.num_subcores
    rows_per_w = seq_len // nw
    chunk = 32
    nchunks = rows_per_w // chunk

    body = functools.partial(_sc_copy_body, mesh.num_cores, rows_per_w,
                             chunk, nchunks)
    sc_copy = pl.kernel(
        body,
        out_type=jax.ShapeDtypeStruct((seq_len, embd_dim), embedding.dtype),
        mesh=mesh,
        scratch_types=[
            pltpu.VMEM((chunk, embd_dim), embedding.dtype),
            pltpu.VMEM((chunk, embd_dim), embedding.dtype),
            pltpu.SemaphoreType.DMA,
            pltpu.SemaphoreType.DMA,
            pltpu.SemaphoreType.DMA,
            pltpu.SemaphoreType.DMA,
        ],
    )
    out = sc_copy(embedding)
    return out[None]


# speedup vs baseline: 47.2471x; 2.3136x over previous
import jax
import jax.numpy as jnp
from jax.experimental import pallas as pl
from jax.experimental.pallas import tpu as pltpu

_NCHUNKS = 8


def _dma_kernel(emb_ref, out_ref, buf, lsem, ssem):
    n = _NCHUNKS
    rows = emb_ref.shape[0] // n
    loads = []
    for i in range(n):
        c = pltpu.make_async_copy(
            emb_ref.at[pl.ds(i * rows, rows)], buf.at[i], lsem.at[i])
        c.start()
        loads.append(c)
    stores = []
    for i in range(n):
        loads[i].wait()
        s = pltpu.make_async_copy(
            buf.at[i], out_ref.at[pl.ds(i * rows, rows)], ssem.at[i])
        s.start()
        stores.append(s)
    for s in stores:
        s.wait()


def kernel(inputs, embedding):
    seq_len, embd_dim = embedding.shape
    out = pl.pallas_call(
        _dma_kernel,
        in_specs=[pl.BlockSpec(memory_space=pl.ANY)],
        out_specs=pl.BlockSpec(memory_space=pl.ANY),
        out_shape=jax.ShapeDtypeStruct((seq_len, embd_dim), embedding.dtype),
        scratch_shapes=[
            pltpu.VMEM((_NCHUNKS, seq_len // _NCHUNKS, embd_dim),
                       embedding.dtype),
            pltpu.SemaphoreType.DMA((_NCHUNKS,)),
            pltpu.SemaphoreType.DMA((_NCHUNKS,)),
        ],
    )(embedding)
    return out[None]


# speedup vs baseline: 47.5913x; 1.0073x over previous
import jax
import jax.numpy as jnp
from jax.experimental import pallas as pl
from jax.experimental.pallas import tpu as pltpu

_NCHUNKS = 4


def _dma_kernel(emb_ref, out_ref, buf, lsem, ssem):
    n = _NCHUNKS
    rows = emb_ref.shape[0] // n
    loads = []
    for i in range(n):
        c = pltpu.make_async_copy(
            emb_ref.at[pl.ds(i * rows, rows)], buf.at[i], lsem.at[i])
        c.start()
        loads.append(c)
    stores = []
    for i in range(n):
        loads[i].wait()
        s = pltpu.make_async_copy(
            buf.at[i], out_ref.at[pl.ds(i * rows, rows)], ssem.at[i])
        s.start()
        stores.append(s)
    for s in stores:
        s.wait()


def kernel(inputs, embedding):
    seq_len, embd_dim = embedding.shape
    out = pl.pallas_call(
        _dma_kernel,
        in_specs=[pl.BlockSpec(memory_space=pl.ANY)],
        out_specs=pl.BlockSpec(memory_space=pl.ANY),
        out_shape=jax.ShapeDtypeStruct((seq_len, embd_dim), embedding.dtype),
        scratch_shapes=[
            pltpu.VMEM((_NCHUNKS, seq_len // _NCHUNKS, embd_dim),
                       embedding.dtype),
            pltpu.SemaphoreType.DMA((_NCHUNKS,)),
            pltpu.SemaphoreType.DMA((_NCHUNKS,)),
        ],
    )(embedding)
    return out[None]
